# Initial kernel scaffold; baseline (speedup 1.0000x reference)
import functools

import jax
import jax.numpy as jnp
from jax import lax
from jax.experimental import pallas as pl
from jax.experimental.pallas import tpu as pltpu
from jax.experimental.pallas import tpu_sc as plsc

NUM_USERS = 25000
NUM_ITEMS = 25000
N = NUM_USERS + NUM_ITEMS
D = 64
E = 800000
N_LAYERS = 3
B = 1024

NC = 2
NS = 16
L = 16

HALF = N // NC
ACC_R = 25088
DUMMY = HALF
ZROWS = 196
CH = 80
EDGES_PER_TILE = E // NS
N_CHUNKS = EDGES_PER_TILE // CH


def _layer_body(emb_hbm, dst_hbm, src_hbm, w_hbm, out_hbm,
                dstv, srcv, wv, liv, rows, zbuf, acc, gsem):
  c = lax.axis_index("c")
  s = lax.axis_index("s")
  lo = c * HALF

  def _zero_zbuf(i, _):
    for k in range(D // L):
      zbuf[i, pl.ds(k * L, L)] = jnp.zeros((L,), jnp.float32)
    return 0
  lax.fori_loop(0, ZROWS, _zero_zbuf, 0)
  zbase = s * (ACC_R // NS)
  for j in range(ACC_R // NS // ZROWS):
    pltpu.sync_copy(zbuf, acc.at[pl.ds(zbase + j * ZROWS, ZROWS)])
  plsc.subcore_barrier()

  def _chunk(g, _):
    base = s * EDGES_PER_TILE + g * CH
    pltpu.sync_copy(dst_hbm.at[pl.ds(base, CH)], dstv)
    pltpu.sync_copy(src_hbm.at[pl.ds(base, CH)], srcv)
    pltpu.sync_copy(w_hbm.at[pl.ds(base, CH)], wv)

    for j in range(CH // L):
      d = dstv[pl.ds(j * L, L)]
      inr = (d >= lo) & (d < lo + HALF)
      liv[pl.ds(j * L, L)] = jnp.where(inr, d - lo, DUMMY)

    pltpu.async_copy(emb_hbm.at[srcv], rows, gsem).wait()

    def _scale(e, _):
      wvec = jnp.full((L,), wv[e], jnp.float32)
      for k in range(D // L):
        rows[e, pl.ds(k * L, L)] = rows[e, pl.ds(k * L, L)] * wvec
      return 0
    lax.fori_loop(0, CH, _scale, 0)

    pltpu.sync_copy(rows, acc.at[liv], add=True)
    return 0
  lax.fori_loop(0, N_CHUNKS, _chunk, 0)
  plsc.subcore_barrier()

  WR = HALF // NS
  pltpu.sync_copy(acc.at[pl.ds(s * WR, WR)], out_hbm.at[pl.ds(lo + s * WR, WR)])

  @pl.when(s == 0)
  def _():
    pltpu.sync_copy(acc.at[pl.ds(NS * WR, HALF - NS * WR)],
                    out_hbm.at[pl.ds(lo + NS * WR, HALF - NS * WR)])


@functools.partial(
    pl.kernel,
    out_type=jax.ShapeDtypeStruct((N, D), jnp.float32),
    mesh=plsc.VectorSubcoreMesh(core_axis_name="c", subcore_axis_name="s"),
    scratch_types=[
        pltpu.VMEM((CH,), jnp.int32),
        pltpu.VMEM((CH,), jnp.int32),
        pltpu.VMEM((CH,), jnp.float32),
        pltpu.VMEM((CH,), jnp.int32),
        pltpu.VMEM((CH, D), jnp.float32),
        pltpu.VMEM((ZROWS, D), jnp.float32),
        pltpu.VMEM_SHARED((ACC_R, D), jnp.float32),
        pltpu.SemaphoreType.DMA,
    ],
    name="lightgcn_layer",
)
def _layer(emb_hbm, dst_hbm, src_hbm, w_hbm, out_hbm, *scratch):
  _layer_body(emb_hbm, dst_hbm, src_hbm, w_hbm, out_hbm, *scratch)


ROWS_PER_TILE_G = B // (NC * NS)


def _gather_body(e0, e1, e2, e3, users_hbm, out_hbm, idxv, b0, b1, obuf, gsem):
  c = lax.axis_index("c")
  s = lax.axis_index("s")
  wid = s * NC + c
  base = wid * ROWS_PER_TILE_G
  pltpu.sync_copy(users_hbm.at[pl.ds(base, ROWS_PER_TILE_G)], idxv)
  pltpu.async_copy(e0.at[idxv], obuf, gsem).wait()
  pltpu.async_copy(e1.at[idxv], b0, gsem).wait()
  pltpu.async_copy(e2.at[idxv], b1, gsem).wait()

  def _addrow(r, _):
    for k in range(D // L):
      sl = pl.ds(k * L, L)
      obuf[r, sl] = obuf[r, sl] + b0[r, sl] + b1[r, sl]
    return 0
  lax.fori_loop(0, ROWS_PER_TILE_G, _addrow, 0)

  pltpu.async_copy(e3.at[idxv], b0, gsem).wait()

  def _addrow2(r, _):
    for k in range(D // L):
      sl = pl.ds(k * L, L)
      obuf[r, sl] = obuf[r, sl] + b0[r, sl]
    return 0
  lax.fori_loop(0, ROWS_PER_TILE_G, _addrow2, 0)

  pltpu.sync_copy(obuf, out_hbm.at[pl.ds(base, ROWS_PER_TILE_G)])


@functools.partial(
    pl.kernel,
    out_type=jax.ShapeDtypeStruct((B, D), jnp.float32),
    mesh=plsc.VectorSubcoreMesh(core_axis_name="c", subcore_axis_name="s"),
    scratch_types=[
        pltpu.VMEM((ROWS_PER_TILE_G,), jnp.int32),
        pltpu.VMEM((ROWS_PER_TILE_G, D), jnp.float32),
        pltpu.VMEM((ROWS_PER_TILE_G, D), jnp.float32),
        pltpu.VMEM((ROWS_PER_TILE_G, D), jnp.float32),
        pltpu.SemaphoreType.DMA,
    ],
    name="lightgcn_user_gather",
)
def _user_gather(e0, e1, e2, e3, users_hbm, out_hbm, *scratch):
  _gather_body(e0, e1, e2, e3, users_hbm, out_hbm, *scratch)


IB = 512


def _rating_kernel(u_ref, i0_ref, i1_ref, i2_ref, i3_ref, o_ref):
  isum = i0_ref[...] + i1_ref[...] + i2_ref[...] + i3_ref[...]
  acc = lax.dot_general(u_ref[...], isum, (((1,), (1,)), ((), ())),
                        preferred_element_type=jnp.float32)
  o_ref[...] = jax.nn.sigmoid(acc * (1.0 / (N_LAYERS + 1) ** 2))


def _rating(u_sum, i0, i1, i2, i3):
  grid = (pl.cdiv(NUM_ITEMS, IB),)
  ispec = pl.BlockSpec((IB, D), lambda n: (n, 0))
  return pl.pallas_call(
      _rating_kernel,
      grid=grid,
      in_specs=[pl.BlockSpec((B, D), lambda n: (0, 0)),
                ispec, ispec, ispec, ispec],
      out_specs=pl.BlockSpec((B, IB), lambda n: (0, n)),
      out_shape=jax.ShapeDtypeStruct((B, NUM_ITEMS), jnp.float32),
      name="lightgcn_rating",
  )(u_sum, i0, i1, i2, i3)


def kernel(user_emb, item_emb, edge_index, edge_weight, users):
  e0 = jnp.concatenate([user_emb, item_emb], axis=0)
  dst = edge_index[0].astype(jnp.int32)
  src = edge_index[1].astype(jnp.int32)
  w = edge_weight.astype(jnp.float32)
  users = users.astype(jnp.int32)

  e1 = _layer(e0, dst, src, w)
  e2 = _layer(e1, dst, src, w)
  e3 = _layer(e2, dst, src, w)

  u_sum = _user_gather(e0, e1, e2, e3, users)
  return _rating(u_sum, e0[NUM_USERS:], e1[NUM_USERS:],
                 e2[NUM_USERS:], e3[NUM_USERS:])


# speedup vs baseline: 2.1260x; 2.1260x over previous
import functools

import jax
import jax.numpy as jnp
from jax import lax
from jax.experimental import pallas as pl
from jax.experimental.pallas import tpu as pltpu
from jax.experimental.pallas import tpu_sc as plsc

NUM_USERS = 25000
NUM_ITEMS = 25000
N = NUM_USERS + NUM_ITEMS
D = 64
E = 800000
N_LAYERS = 3
B = 1024

NC = 2
NS = 16
L = 16

HALF = N // NC
ACC_R = 25088
DUMMY = HALF
ZROWS = 224
CH = 80
EDGES_PER_TILE = E // NS
N_CHUNKS = EDGES_PER_TILE // CH


def _layer_body(emb_hbm, dst_hbm, src_hbm, w_hbm, out_hbm,
                dstv, srcv, wv, liv, rows, zbuf, acc, gsem):
  c = lax.axis_index("c")
  s = lax.axis_index("s")
  lo = c * HALF

  def _zero_zbuf(i, _):
    for k in range(D // L):
      zbuf[i, pl.ds(k * L, L)] = jnp.zeros((L,), jnp.float32)
    return 0
  lax.fori_loop(0, ZROWS, _zero_zbuf, 0)
  zbase = s * (ACC_R // NS)
  for j in range(ACC_R // NS // ZROWS):
    pltpu.sync_copy(zbuf, acc.at[pl.ds(zbase + j * ZROWS, ZROWS)])
  plsc.subcore_barrier()

  def _chunk(g, _):
    base = s * EDGES_PER_TILE + g * CH
    pltpu.sync_copy(dst_hbm.at[pl.ds(base, CH)], dstv)
    pltpu.sync_copy(src_hbm.at[pl.ds(base, CH)], srcv)
    pltpu.sync_copy(w_hbm.at[pl.ds(base, CH)], wv)

    for j in range(CH // L):
      d = dstv[pl.ds(j * L, L)]
      inr = (d >= lo) & (d < lo + HALF)
      liv[pl.ds(j * L, L)] = jnp.where(inr, d - lo, DUMMY)

    pltpu.async_copy(emb_hbm.at[srcv], rows, gsem).wait()

    def _scale(j, _):
      wall = wv[pl.ds(j * L, L)]
      for e16 in range(L):
        wvec = jnp.full((L,), wall[e16], jnp.float32)
        e = j * L + e16
        for k in range(D // L):
          sl = pl.ds(k * L, L)
          rows[e, sl] = rows[e, sl] * wvec
      return 0
    lax.fori_loop(0, CH // L, _scale, 0)

    pltpu.sync_copy(rows, acc.at[liv], add=True)
    return 0
  lax.fori_loop(0, N_CHUNKS, _chunk, 0)
  plsc.subcore_barrier()

  WR = 1560
  pltpu.sync_copy(acc.at[pl.ds(s * WR, WR)], out_hbm.at[pl.ds(lo + s * WR, WR)])

  @pl.when(s == 0)
  def _():
    pltpu.sync_copy(acc.at[pl.ds(NS * WR, HALF - NS * WR)],
                    out_hbm.at[pl.ds(lo + NS * WR, HALF - NS * WR)])


@functools.partial(
    pl.kernel,
    out_type=jax.ShapeDtypeStruct((N, D), jnp.float32),
    mesh=plsc.VectorSubcoreMesh(core_axis_name="c", subcore_axis_name="s"),
    scratch_types=[
        pltpu.VMEM((CH,), jnp.int32),
        pltpu.VMEM((CH,), jnp.int32),
        pltpu.VMEM((CH,), jnp.float32),
        pltpu.VMEM((CH,), jnp.int32),
        pltpu.VMEM((CH, D), jnp.float32),
        pltpu.VMEM((ZROWS, D), jnp.float32),
        pltpu.VMEM_SHARED((ACC_R, D), jnp.float32),
        pltpu.SemaphoreType.DMA,
    ],
    compiler_params=pltpu.CompilerParams(use_tc_tiling_on_sc=False),
    name="lightgcn_layer",
)
def _layer(emb_hbm, dst_hbm, src_hbm, w_hbm, out_hbm, *scratch):
  _layer_body(emb_hbm, dst_hbm, src_hbm, w_hbm, out_hbm, *scratch)


ROWS_PER_TILE_G = B // (NC * NS)


def _gather_body(e0, e1, e2, e3, users_hbm, out_hbm, idxv, b0, b1, obuf, gsem):
  c = lax.axis_index("c")
  s = lax.axis_index("s")
  wid = s * NC + c
  base = wid * ROWS_PER_TILE_G
  pltpu.sync_copy(users_hbm.at[pl.ds(base, ROWS_PER_TILE_G)], idxv)
  pltpu.async_copy(e0.at[idxv], obuf, gsem).wait()
  pltpu.async_copy(e1.at[idxv], b0, gsem).wait()
  pltpu.async_copy(e2.at[idxv], b1, gsem).wait()

  def _addrow(r, _):
    for k in range(D // L):
      sl = pl.ds(k * L, L)
      obuf[r, sl] = obuf[r, sl] + b0[r, sl] + b1[r, sl]
    return 0
  lax.fori_loop(0, ROWS_PER_TILE_G, _addrow, 0)

  pltpu.async_copy(e3.at[idxv], b0, gsem).wait()

  def _addrow2(r, _):
    for k in range(D // L):
      sl = pl.ds(k * L, L)
      obuf[r, sl] = obuf[r, sl] + b0[r, sl]
    return 0
  lax.fori_loop(0, ROWS_PER_TILE_G, _addrow2, 0)

  pltpu.sync_copy(obuf, out_hbm.at[pl.ds(base, ROWS_PER_TILE_G)])


@functools.partial(
    pl.kernel,
    out_type=jax.ShapeDtypeStruct((B, D), jnp.float32),
    mesh=plsc.VectorSubcoreMesh(core_axis_name="c", subcore_axis_name="s"),
    scratch_types=[
        pltpu.VMEM((ROWS_PER_TILE_G,), jnp.int32),
        pltpu.VMEM((ROWS_PER_TILE_G, D), jnp.float32),
        pltpu.VMEM((ROWS_PER_TILE_G, D), jnp.float32),
        pltpu.VMEM((ROWS_PER_TILE_G, D), jnp.float32),
        pltpu.SemaphoreType.DMA,
    ],
    compiler_params=pltpu.CompilerParams(use_tc_tiling_on_sc=False),
    name="lightgcn_user_gather",
)
def _user_gather(e0, e1, e2, e3, users_hbm, out_hbm, *scratch):
  _gather_body(e0, e1, e2, e3, users_hbm, out_hbm, *scratch)


IB = 512


def _rating_kernel(u_ref, i0_ref, i1_ref, i2_ref, i3_ref, o_ref):
  isum = i0_ref[...] + i1_ref[...] + i2_ref[...] + i3_ref[...]
  acc = lax.dot_general(u_ref[...], isum, (((1,), (1,)), ((), ())),
                        preferred_element_type=jnp.float32)
  o_ref[...] = jax.nn.sigmoid(acc * (1.0 / (N_LAYERS + 1) ** 2))


def _rating(u_sum, i0, i1, i2, i3):
  grid = (pl.cdiv(NUM_ITEMS, IB),)
  ispec = pl.BlockSpec((IB, D), lambda n: (n, 0))
  return pl.pallas_call(
      _rating_kernel,
      grid=grid,
      in_specs=[pl.BlockSpec((B, D), lambda n: (0, 0)),
                ispec, ispec, ispec, ispec],
      out_specs=pl.BlockSpec((B, IB), lambda n: (0, n)),
      out_shape=jax.ShapeDtypeStruct((B, NUM_ITEMS), jnp.float32),
      name="lightgcn_rating",
  )(u_sum, i0, i1, i2, i3)


def kernel(user_emb, item_emb, edge_index, edge_weight, users):
  e0 = jnp.concatenate([user_emb, item_emb], axis=0)
  dst = edge_index[0].astype(jnp.int32)
  src = edge_index[1].astype(jnp.int32)
  w = edge_weight.astype(jnp.float32)
  users = users.astype(jnp.int32)

  e1 = _layer(e0, dst, src, w)
  e2 = _layer(e1, dst, src, w)
  e3 = _layer(e2, dst, src, w)

  u_sum = _user_gather(e0, e1, e2, e3, users)
  return _rating(u_sum, e0[NUM_USERS:], e1[NUM_USERS:],
                 e2[NUM_USERS:], e3[NUM_USERS:])
